# baseline (speedup 1.0000x reference)
import functools

import jax
import jax.numpy as jnp
from jax import lax
from jax.experimental import pallas as pl
from jax.experimental.pallas import tpu as pltpu
from jax.experimental.pallas import tpu_sc as plsc

T = 2048
DIM = 2048
INTER = 1408
E = 8
TM = 256
NP = T * 2 + E * TM
W = NP // TM
TMG = 256
NW = 32
CH = 16
CPW = T // (NW * CH)
CHC = 8
CPC = T // (NW * CHC)


def _gate_kernel(x_ref, gw_ref, meta_ref, tot_ref, carry_ref):
    i = pl.program_id(0)

    @pl.when(i == 0)
    def _init():
        carry_ref[...] = jnp.zeros_like(carry_ref)

    s = jax.nn.sigmoid(
        lax.dot_general(x_ref[...], gw_ref[...], (((1,), (1,)), ((), ())),
                        preferred_element_type=jnp.float32))
    eidx = lax.broadcasted_iota(jnp.int32, s.shape, 1)
    big = jnp.int32(E + 1)
    m1 = jnp.max(s, axis=-1, keepdims=True)
    i1 = jnp.min(jnp.where(s == m1, eidx, big), axis=-1, keepdims=True)
    sel1 = eidx == i1
    s2 = jnp.where(sel1, -jnp.inf, s)
    m2 = jnp.max(s2, axis=-1, keepdims=True)
    i2 = jnp.min(jnp.where(s2 == m2, eidx, big), axis=-1, keepdims=True)
    sel2 = eidx == i2
    denom = m1 + m2
    onehot = jnp.where(sel1 | sel2, 1.0, 0.0)

    ri = lax.broadcasted_iota(jnp.int32, (TMG, TMG), 0)
    ci = lax.broadcasted_iota(jnp.int32, (TMG, TMG), 1)
    ltri = jnp.where(ci < ri, 1.0, 0.0)
    cnt = lax.dot_general(ltri, onehot, (((1,), (0,)), ((), ())),
                          preferred_element_type=jnp.float32) + carry_ref[...]

    ef = eidx.astype(jnp.float32)
    e0 = jnp.sum(jnp.where(sel1, ef, 0.0), axis=-1)
    e1 = jnp.sum(jnp.where(sel2, ef, 0.0), axis=-1)
    c0 = jnp.sum(jnp.where(sel1, cnt, 0.0), axis=-1)
    c1 = jnp.sum(jnp.where(sel2, cnt, 0.0), axis=-1)
    w0 = (m1 / denom)[:, 0]
    w1 = (m2 / denom)[:, 0]
    z = jnp.zeros_like(w0)
    meta_ref[...] = jnp.stack([e0, e1, w0, w1, c0, c1, z, z], axis=0)

    newcarry = carry_ref[...] + jnp.sum(onehot, axis=0, keepdims=True)
    carry_ref[...] = newcarry
    tot_ref[...] = newcarry


def _prefetch_weights(m, gm_ref, pos_ref, nxt_ref, hbm_refs, bufs, sems):
    pos = pos_ref[m]
    slot = lax.rem(pos, 2)

    @pl.when(m == 0)
    def _prime():
        g0 = gm_ref[0]
        nx = nxt_ref[0]
        for k, (hbm, buf) in enumerate(zip(hbm_refs, bufs)):
            pltpu.make_async_copy(hbm.at[g0], buf.at[0], sems.at[0, k]).start()

        @pl.when(nx >= 0)
        def _():
            for k, (hbm, buf) in enumerate(zip(hbm_refs, bufs)):
                pltpu.make_async_copy(hbm.at[nx], buf.at[1],
                                      sems.at[1, k]).start()

    first = (m == 0) | (pos != pos_ref[jnp.maximum(m - 1, 0)])

    @pl.when(first)
    def _wait_issue():
        g = gm_ref[m]
        for k, (hbm, buf) in enumerate(zip(hbm_refs, bufs)):
            pltpu.make_async_copy(hbm.at[g], buf.at[slot],
                                  sems.at[slot, k]).wait()
        nx = nxt_ref[m]

        @pl.when((nx >= 0) & (m != 0))
        def _():
            for k, (hbm, buf) in enumerate(zip(hbm_refs, bufs)):
                pltpu.make_async_copy(hbm.at[nx], buf.at[1 - slot],
                                      sems.at[1 - slot, k]).start()

    return slot


def _stage_a_kernel(gm_ref, vl_ref, pos_ref, nxt_ref, xs_ref, w1_hbm, b1_ref,
                    w3_hbm, b3_ref, h_ref, w1buf, w3buf, sems):
    m = pl.program_id(0)
    slot = _prefetch_weights(m, gm_ref, pos_ref, nxt_ref,
                             [w1_hbm, w3_hbm], [w1buf, w3buf], sems)

    @pl.when(vl_ref[m] == 1)
    def _compute():
        xb = xs_ref[...]
        h1 = lax.dot_general(xb, w1buf[slot], (((1,), (1,)), ((), ())),
                             preferred_element_type=jnp.float32) + b1_ref[0]
        h3 = lax.dot_general(xb, w3buf[slot], (((1,), (1,)), ((), ())),
                             preferred_element_type=jnp.float32) + b3_ref[0]
        h_ref[...] = (h1 * jax.nn.sigmoid(h1)) * h3


def _stage_b_kernel(gm_ref, vl_ref, pos_ref, nxt_ref, h_ref, w2_hbm, b2_ref,
                    ws_ref, out_ref, w2buf, sems):
    m = pl.program_id(0)
    slot = _prefetch_weights(m, gm_ref, pos_ref, nxt_ref,
                             [w2_hbm], [w2buf], sems)

    @pl.when(vl_ref[m] == 1)
    def _compute():
        out_ref[...] = (lax.dot_general(
            h_ref[...], w2buf[slot], (((1,), (1,)), ((), ())),
            preferred_element_type=jnp.float32) + b2_ref[0]) * ws_ref[...]


def _dispatch_body(x_hbm, meta_hbm, aoff_hbm, xs_hbm, dest_hbm, ws_hbm,
                   a_vm, f0, f1, f4, f5, idx0, idx1, rows,
                   eb, cb, wb, wsbuf, sem):
    wid = lax.axis_index("s") * 2 + lax.axis_index("c")
    pltpu.sync_copy(aoff_hbm, a_vm)

    @pl.when(wid == 0)
    def _build_ws():
        for slot in range(2):
            pltpu.sync_copy(meta_hbm.at[slot, :], eb)
            pltpu.sync_copy(meta_hbm.at[4 + slot, :], cb)
            pltpu.sync_copy(meta_hbm.at[2 + slot, :], wb)

            def _scat(j, _):
                ts = pl.ds(j * 16, 16)
                ev = eb[ts].astype(jnp.int32)
                dv = plsc.load_gather(a_vm, [ev]) + cb[ts].astype(jnp.int32)
                plsc.store_scatter(wsbuf, [dv], wb[ts])
                return _

            lax.fori_loop(0, T // 16, _scat, 0)
        pltpu.sync_copy(wsbuf, ws_hbm)

    for ch in range(CPW):
        tok = (wid * CPW + ch) * CH
        pltpu.sync_copy(meta_hbm.at[0, pl.ds(tok, CH)], f0)
        pltpu.sync_copy(meta_hbm.at[1, pl.ds(tok, CH)], f1)
        pltpu.sync_copy(meta_hbm.at[4, pl.ds(tok, CH)], f4)
        pltpu.sync_copy(meta_hbm.at[5, pl.ds(tok, CH)], f5)
        e0 = f0[...].astype(jnp.int32)
        e1 = f1[...].astype(jnp.int32)
        a0 = plsc.load_gather(a_vm, [e0])
        a1 = plsc.load_gather(a_vm, [e1])
        idx0[...] = a0 + f4[...].astype(jnp.int32)
        idx1[...] = a1 + f5[...].astype(jnp.int32)
        pltpu.sync_copy(x_hbm.at[pl.ds(tok, CH)], rows)
        pltpu.async_copy(rows, xs_hbm.at[idx0], sem).wait()
        pltpu.async_copy(rows, xs_hbm.at[idx1], sem).wait()
        pltpu.sync_copy(idx0, dest_hbm.at[0, pl.ds(tok, CH)])
        pltpu.sync_copy(idx1, dest_hbm.at[1, pl.ds(tok, CH)])


def _combine_body(out_hbm, dest_hbm, y_hbm,
                  idx0a, idx1a, b0a, b1a, idx0b, idx1b, b0b, b1b,
                  sema, semb):
    wid = lax.axis_index("s") * 2 + lax.axis_index("c")
    slots = ((idx0a, idx1a, b0a, b1a, sema), (idx0b, idx1b, b0b, b1b, semb))

    def _load(ch, slot):
        i0, i1, b0, b1, sm = slots[slot]
        tok = (wid * CPC + ch) * CHC
        pltpu.sync_copy(dest_hbm.at[0, pl.ds(tok, CHC)], i0)
        pltpu.sync_copy(dest_hbm.at[1, pl.ds(tok, CHC)], i1)
        pltpu.async_copy(out_hbm.at[i0], b0, sm)
        pltpu.async_copy(out_hbm.at[i1], b1, sm)

    _load(0, 0)
    for ch in range(CPC):
        cur = ch % 2
        if ch + 1 < CPC:
            _load(ch + 1, 1 - cur)
        i0, i1, b0, b1, sm = slots[cur]
        tok = (wid * CPC + ch) * CHC
        pltpu.make_async_copy(out_hbm.at[i0], b0, sm).wait()
        pltpu.make_async_copy(out_hbm.at[i1], b1, sm).wait()

        def _row(i, _, b0=b0, b1=b1):
            def _col(j, _):
                for u in range(4):
                    cs = pl.ds(j * 64 + u * 16, 16)
                    b0[i, cs] = b0[i, cs] + b1[i, cs]
                return _

            return lax.fori_loop(0, DIM // 64, _col, _)

        lax.fori_loop(0, CHC, _row, 0)
        pltpu.sync_copy(b0, y_hbm.at[pl.ds(tok, CHC)])


def kernel(x, gate_w, w1, b1, w2, b2, w3, b3):
    meta, totals = pl.pallas_call(
        _gate_kernel,
        grid=(T // TMG,),
        in_specs=[
            pl.BlockSpec((TMG, DIM), lambda i: (i, 0)),
            pl.BlockSpec((E, DIM), lambda i: (0, 0)),
        ],
        out_specs=[
            pl.BlockSpec((E, TMG), lambda i: (0, i)),
            pl.BlockSpec((1, E), lambda i: (0, 0)),
        ],
        out_shape=[
            jax.ShapeDtypeStruct((E, T), jnp.float32),
            jax.ShapeDtypeStruct((1, E), jnp.float32),
        ],
        scratch_shapes=[pltpu.VMEM((1, E), jnp.float32)],
    )(x, gate_w)

    counts = totals[0].astype(jnp.int32)
    starts = []
    cur = jnp.int32(0)
    for e in range(E):
        starts.append(cur)
        cur = ((cur + counts[e] + TM - 1) // TM) * TM
    aoff = jnp.stack(starts)
    rows0 = jnp.arange(W, dtype=jnp.int32) * TM
    gmap = jnp.clip(jnp.searchsorted(aoff, rows0, side="right").astype(jnp.int32) - 1,
                    0, E - 1)
    valid = (rows0 < aoff[gmap] + counts[gmap]).astype(jnp.int32)
    change = jnp.concatenate([jnp.ones((1,), jnp.int32),
                              (gmap[1:] != gmap[:-1]).astype(jnp.int32)])
    posm = jnp.cumsum(change).astype(jnp.int32) - 1
    nidx = jnp.searchsorted(gmap, gmap, side="right").astype(jnp.int32)
    nxt = jnp.where(nidx < W, gmap[jnp.clip(nidx, 0, W - 1)], -1).astype(jnp.int32)

    mesh = plsc.VectorSubcoreMesh(core_axis_name="c", subcore_axis_name="s")
    xs, dest, ws = pl.kernel(
        _dispatch_body,
        out_type=[
            jax.ShapeDtypeStruct((NP, DIM), jnp.float32),
            jax.ShapeDtypeStruct((2, T), jnp.int32),
            jax.ShapeDtypeStruct((NP,), jnp.float32),
        ],
        mesh=mesh,
        scratch_types=[
            pltpu.VMEM((E,), jnp.int32),
            pltpu.VMEM((CH,), jnp.float32),
            pltpu.VMEM((CH,), jnp.float32),
            pltpu.VMEM((CH,), jnp.float32),
            pltpu.VMEM((CH,), jnp.float32),
            pltpu.VMEM((CH,), jnp.int32),
            pltpu.VMEM((CH,), jnp.int32),
            pltpu.VMEM((CH, DIM), jnp.float32),
            pltpu.VMEM((T,), jnp.float32),
            pltpu.VMEM((T,), jnp.float32),
            pltpu.VMEM((T,), jnp.float32),
            pltpu.VMEM((NP,), jnp.float32),
            pltpu.SemaphoreType.DMA,
        ],
        compiler_params=pltpu.CompilerParams(needs_layout_passes=False),
    )(x, meta, aoff)

    hmid = pl.pallas_call(
        _stage_a_kernel,
        grid_spec=pltpu.PrefetchScalarGridSpec(
            num_scalar_prefetch=4,
            grid=(W,),
            in_specs=[
                pl.BlockSpec((TM, DIM), lambda m, gm, vl, po, nx: (m, 0)),
                pl.BlockSpec(memory_space=pl.ANY),
                pl.BlockSpec((1, 1, INTER), lambda m, gm, vl, po, nx: (gm[m], 0, 0)),
                pl.BlockSpec(memory_space=pl.ANY),
                pl.BlockSpec((1, 1, INTER), lambda m, gm, vl, po, nx: (gm[m], 0, 0)),
            ],
            out_specs=pl.BlockSpec((TM, INTER), lambda m, gm, vl, po, nx: (m, 0)),
            scratch_shapes=[
                pltpu.VMEM((2, INTER, DIM), jnp.float32),
                pltpu.VMEM((2, INTER, DIM), jnp.float32),
                pltpu.SemaphoreType.DMA((2, 2)),
            ],
        ),
        out_shape=jax.ShapeDtypeStruct((NP, INTER), jnp.float32),
        compiler_params=pltpu.CompilerParams(
            dimension_semantics=("arbitrary",)),
    )(gmap, valid, posm, nxt, xs, w1, b1[:, None, :], w3, b3[:, None, :])

    outm = pl.pallas_call(
        _stage_b_kernel,
        grid_spec=pltpu.PrefetchScalarGridSpec(
            num_scalar_prefetch=4,
            grid=(W,),
            in_specs=[
                pl.BlockSpec((TM, INTER), lambda m, gm, vl, po, nx: (m, 0)),
                pl.BlockSpec(memory_space=pl.ANY),
                pl.BlockSpec((1, 1, DIM), lambda m, gm, vl, po, nx: (gm[m], 0, 0)),
                pl.BlockSpec((TM, 1), lambda m, gm, vl, po, nx: (m, 0)),
            ],
            out_specs=pl.BlockSpec((TM, DIM), lambda m, gm, vl, po, nx: (m, 0)),
            scratch_shapes=[
                pltpu.VMEM((2, DIM, INTER), jnp.float32),
                pltpu.SemaphoreType.DMA((2, 1)),
            ],
        ),
        out_shape=jax.ShapeDtypeStruct((NP, DIM), jnp.float32),
        compiler_params=pltpu.CompilerParams(
            dimension_semantics=("arbitrary",)),
    )(gmap, valid, posm, nxt, hmid, w2, b2[:, None, :], ws[:, None])

    y = pl.kernel(
        _combine_body,
        out_type=jax.ShapeDtypeStruct((T, DIM), jnp.float32),
        mesh=plsc.VectorSubcoreMesh(core_axis_name="c", subcore_axis_name="s"),
        scratch_types=[
            pltpu.VMEM((CHC,), jnp.int32),
            pltpu.VMEM((CHC,), jnp.int32),
            pltpu.VMEM((CHC, DIM), jnp.float32),
            pltpu.VMEM((CHC, DIM), jnp.float32),
            pltpu.VMEM((CHC,), jnp.int32),
            pltpu.VMEM((CHC,), jnp.int32),
            pltpu.VMEM((CHC, DIM), jnp.float32),
            pltpu.VMEM((CHC, DIM), jnp.float32),
            pltpu.SemaphoreType.DMA,
            pltpu.SemaphoreType.DMA,
        ],
        compiler_params=pltpu.CompilerParams(needs_layout_passes=False),
    )(outm, dest)
    return y


# speedup vs baseline: 1.0750x; 1.0750x over previous
import functools

import jax
import jax.numpy as jnp
from jax import lax
from jax.experimental import pallas as pl
from jax.experimental.pallas import tpu as pltpu
from jax.experimental.pallas import tpu_sc as plsc

T = 2048
DIM = 2048
INTER = 1408
E = 8
TM = 256
NP = T * 2 + E * TM
W = NP // TM
TMG = 256
NW = 32
CH = 16
CPW = T // (NW * CH)


def _gate_kernel(x_ref, gw_ref, meta_ref, tot_ref, carry_ref):
    i = pl.program_id(0)

    @pl.when(i == 0)
    def _init():
        carry_ref[...] = jnp.zeros_like(carry_ref)

    s = jax.nn.sigmoid(
        lax.dot_general(x_ref[...], gw_ref[...], (((1,), (1,)), ((), ())),
                        preferred_element_type=jnp.float32))
    eidx = lax.broadcasted_iota(jnp.int32, s.shape, 1)
    big = jnp.int32(E + 1)
    m1 = jnp.max(s, axis=-1, keepdims=True)
    i1 = jnp.min(jnp.where(s == m1, eidx, big), axis=-1, keepdims=True)
    sel1 = eidx == i1
    s2 = jnp.where(sel1, -jnp.inf, s)
    m2 = jnp.max(s2, axis=-1, keepdims=True)
    i2 = jnp.min(jnp.where(s2 == m2, eidx, big), axis=-1, keepdims=True)
    sel2 = eidx == i2
    denom = m1 + m2
    onehot = jnp.where(sel1 | sel2, 1.0, 0.0)

    ri = lax.broadcasted_iota(jnp.int32, (TMG, TMG), 0)
    ci = lax.broadcasted_iota(jnp.int32, (TMG, TMG), 1)
    ltri = jnp.where(ci < ri, 1.0, 0.0)
    cnt = lax.dot_general(ltri, onehot, (((1,), (0,)), ((), ())),
                          preferred_element_type=jnp.float32) + carry_ref[...]

    ef = eidx.astype(jnp.float32)
    e0 = jnp.sum(jnp.where(sel1, ef, 0.0), axis=-1)
    e1 = jnp.sum(jnp.where(sel2, ef, 0.0), axis=-1)
    c0 = jnp.sum(jnp.where(sel1, cnt, 0.0), axis=-1)
    c1 = jnp.sum(jnp.where(sel2, cnt, 0.0), axis=-1)
    w0 = (m1 / denom)[:, 0]
    w1 = (m2 / denom)[:, 0]
    z = jnp.zeros_like(w0)
    meta_ref[...] = jnp.stack([e0, e1, w0, w1, c0, c1, z, z], axis=0)

    newcarry = carry_ref[...] + jnp.sum(onehot, axis=0, keepdims=True)
    carry_ref[...] = newcarry
    tot_ref[...] = newcarry


def _prefetch_weights(m, gm_ref, pos_ref, nxt_ref, hbm_refs, bufs, sems):
    pos = pos_ref[m]
    slot = lax.rem(pos, 2)

    @pl.when(m == 0)
    def _prime():
        g0 = gm_ref[0]
        nx = nxt_ref[0]
        for k, (hbm, buf) in enumerate(zip(hbm_refs, bufs)):
            pltpu.make_async_copy(hbm.at[g0], buf.at[0], sems.at[0, k]).start()

        @pl.when(nx >= 0)
        def _():
            for k, (hbm, buf) in enumerate(zip(hbm_refs, bufs)):
                pltpu.make_async_copy(hbm.at[nx], buf.at[1],
                                      sems.at[1, k]).start()

    first = (m == 0) | (pos != pos_ref[jnp.maximum(m - 1, 0)])

    @pl.when(first)
    def _wait_issue():
        g = gm_ref[m]
        for k, (hbm, buf) in enumerate(zip(hbm_refs, bufs)):
            pltpu.make_async_copy(hbm.at[g], buf.at[slot],
                                  sems.at[slot, k]).wait()
        nx = nxt_ref[m]

        @pl.when((nx >= 0) & (m != 0))
        def _():
            for k, (hbm, buf) in enumerate(zip(hbm_refs, bufs)):
                pltpu.make_async_copy(hbm.at[nx], buf.at[1 - slot],
                                      sems.at[1 - slot, k]).start()

    return slot


def _stage_a_kernel(gm_ref, vl_ref, pos_ref, nxt_ref, xs_ref, w1_hbm, b1_ref,
                    w3_hbm, b3_ref, h_ref, w1buf, w3buf, sems):
    m = pl.program_id(0)
    slot = _prefetch_weights(m, gm_ref, pos_ref, nxt_ref,
                             [w1_hbm, w3_hbm], [w1buf, w3buf], sems)

    @pl.when(vl_ref[m] == 1)
    def _compute():
        xb = xs_ref[...]
        h1 = lax.dot_general(xb, w1buf[slot], (((1,), (1,)), ((), ())),
                             preferred_element_type=jnp.float32) + b1_ref[0]
        h3 = lax.dot_general(xb, w3buf[slot], (((1,), (1,)), ((), ())),
                             preferred_element_type=jnp.float32) + b3_ref[0]
        h_ref[...] = (h1 * jax.nn.sigmoid(h1)) * h3


def _stage_b_kernel(gm_ref, vl_ref, pos_ref, nxt_ref, h_ref, w2_hbm, b2_ref,
                    ws_ref, out_ref, w2buf, sems):
    m = pl.program_id(0)
    slot = _prefetch_weights(m, gm_ref, pos_ref, nxt_ref,
                             [w2_hbm], [w2buf], sems)

    @pl.when(vl_ref[m] == 1)
    def _compute():
        out_ref[...] = (lax.dot_general(
            h_ref[...], w2buf[slot], (((1,), (1,)), ((), ())),
            preferred_element_type=jnp.float32) + b2_ref[0]) * ws_ref[...]


def _dispatch_body(x_hbm, meta_hbm, aoff_hbm, xs_hbm, dest_hbm, ws_hbm,
                   a_vm, f0, f1, f4, f5, idx0, idx1, rows,
                   eb, cb, wb, wsbuf, sem):
    wid = lax.axis_index("s") * 2 + lax.axis_index("c")
    pltpu.sync_copy(aoff_hbm, a_vm)

    @pl.when(wid == 0)
    def _build_ws():
        for slot in range(2):
            pltpu.sync_copy(meta_hbm.at[slot, :], eb)
            pltpu.sync_copy(meta_hbm.at[4 + slot, :], cb)
            pltpu.sync_copy(meta_hbm.at[2 + slot, :], wb)

            def _scat(j, _):
                ts = pl.ds(j * 16, 16)
                ev = eb[ts].astype(jnp.int32)
                dv = plsc.load_gather(a_vm, [ev]) + cb[ts].astype(jnp.int32)
                plsc.store_scatter(wsbuf, [dv], wb[ts])
                return _

            lax.fori_loop(0, T // 16, _scat, 0)
        pltpu.sync_copy(wsbuf, ws_hbm)

    for ch in range(CPW):
        tok = (wid * CPW + ch) * CH
        pltpu.sync_copy(meta_hbm.at[0, pl.ds(tok, CH)], f0)
        pltpu.sync_copy(meta_hbm.at[1, pl.ds(tok, CH)], f1)
        pltpu.sync_copy(meta_hbm.at[4, pl.ds(tok, CH)], f4)
        pltpu.sync_copy(meta_hbm.at[5, pl.ds(tok, CH)], f5)
        e0 = f0[...].astype(jnp.int32)
        e1 = f1[...].astype(jnp.int32)
        a0 = plsc.load_gather(a_vm, [e0])
        a1 = plsc.load_gather(a_vm, [e1])
        idx0[...] = a0 + f4[...].astype(jnp.int32)
        idx1[...] = a1 + f5[...].astype(jnp.int32)
        pltpu.sync_copy(x_hbm.at[pl.ds(tok, CH)], rows)
        pltpu.async_copy(rows, xs_hbm.at[idx0], sem).wait()
        pltpu.async_copy(rows, xs_hbm.at[idx1], sem).wait()
        pltpu.sync_copy(idx0, dest_hbm.at[0, pl.ds(tok, CH)])
        pltpu.sync_copy(idx1, dest_hbm.at[1, pl.ds(tok, CH)])


def _combine_body(out_hbm, dest_hbm, y_hbm,
                  idx0, idx1, buf0, buf1, sem):
    wid = lax.axis_index("s") * 2 + lax.axis_index("c")
    for ch in range(CPW):
        tok = (wid * CPW + ch) * CH
        pltpu.sync_copy(dest_hbm.at[0, pl.ds(tok, CH)], idx0)
        pltpu.sync_copy(dest_hbm.at[1, pl.ds(tok, CH)], idx1)
        pltpu.async_copy(out_hbm.at[idx0], buf0, sem).wait()
        pltpu.async_copy(out_hbm.at[idx1], buf1, sem).wait()

        def _row(i, _):
            def _col(j, _):
                for u in range(4):
                    cs = pl.ds(j * 64 + u * 16, 16)
                    buf0[i, cs] = buf0[i, cs] + buf1[i, cs]
                return _

            return lax.fori_loop(0, DIM // 64, _col, _)

        lax.fori_loop(0, CH, _row, 0)
        pltpu.sync_copy(buf0, y_hbm.at[pl.ds(tok, CH)])


def kernel(x, gate_w, w1, b1, w2, b2, w3, b3):
    meta, totals = pl.pallas_call(
        _gate_kernel,
        grid=(T // TMG,),
        in_specs=[
            pl.BlockSpec((TMG, DIM), lambda i: (i, 0)),
            pl.BlockSpec((E, DIM), lambda i: (0, 0)),
        ],
        out_specs=[
            pl.BlockSpec((E, TMG), lambda i: (0, i)),
            pl.BlockSpec((1, E), lambda i: (0, 0)),
        ],
        out_shape=[
            jax.ShapeDtypeStruct((E, T), jnp.float32),
            jax.ShapeDtypeStruct((1, E), jnp.float32),
        ],
        scratch_shapes=[pltpu.VMEM((1, E), jnp.float32)],
    )(x, gate_w)

    counts = totals[0].astype(jnp.int32)
    starts = []
    cur = jnp.int32(0)
    for e in range(E):
        starts.append(cur)
        cur = ((cur + counts[e] + TM - 1) // TM) * TM
    aoff = jnp.stack(starts)
    rows0 = jnp.arange(W, dtype=jnp.int32) * TM
    gmap = jnp.clip(jnp.searchsorted(aoff, rows0, side="right").astype(jnp.int32) - 1,
                    0, E - 1)
    valid = (rows0 < aoff[gmap] + counts[gmap]).astype(jnp.int32)
    change = jnp.concatenate([jnp.ones((1,), jnp.int32),
                              (gmap[1:] != gmap[:-1]).astype(jnp.int32)])
    posm = jnp.cumsum(change).astype(jnp.int32) - 1
    nidx = jnp.searchsorted(gmap, gmap, side="right").astype(jnp.int32)
    nxt = jnp.where(nidx < W, gmap[jnp.clip(nidx, 0, W - 1)], -1).astype(jnp.int32)

    mesh = plsc.VectorSubcoreMesh(core_axis_name="c", subcore_axis_name="s")
    xs, dest, ws = pl.kernel(
        _dispatch_body,
        out_type=[
            jax.ShapeDtypeStruct((NP, DIM), jnp.float32),
            jax.ShapeDtypeStruct((2, T), jnp.int32),
            jax.ShapeDtypeStruct((NP,), jnp.float32),
        ],
        mesh=mesh,
        scratch_types=[
            pltpu.VMEM((E,), jnp.int32),
            pltpu.VMEM((CH,), jnp.float32),
            pltpu.VMEM((CH,), jnp.float32),
            pltpu.VMEM((CH,), jnp.float32),
            pltpu.VMEM((CH,), jnp.float32),
            pltpu.VMEM((CH,), jnp.int32),
            pltpu.VMEM((CH,), jnp.int32),
            pltpu.VMEM((CH, DIM), jnp.float32),
            pltpu.VMEM((T,), jnp.float32),
            pltpu.VMEM((T,), jnp.float32),
            pltpu.VMEM((T,), jnp.float32),
            pltpu.VMEM((NP,), jnp.float32),
            pltpu.SemaphoreType.DMA,
        ],
        compiler_params=pltpu.CompilerParams(needs_layout_passes=False),
    )(x, meta, aoff)

    hmid = pl.pallas_call(
        _stage_a_kernel,
        grid_spec=pltpu.PrefetchScalarGridSpec(
            num_scalar_prefetch=4,
            grid=(W,),
            in_specs=[
                pl.BlockSpec((TM, DIM), lambda m, gm, vl, po, nx: (m, 0)),
                pl.BlockSpec(memory_space=pl.ANY),
                pl.BlockSpec((1, 1, INTER), lambda m, gm, vl, po, nx: (gm[m], 0, 0)),
                pl.BlockSpec(memory_space=pl.ANY),
                pl.BlockSpec((1, 1, INTER), lambda m, gm, vl, po, nx: (gm[m], 0, 0)),
            ],
            out_specs=pl.BlockSpec((TM, INTER), lambda m, gm, vl, po, nx: (m, 0)),
            scratch_shapes=[
                pltpu.VMEM((2, INTER, DIM), jnp.float32),
                pltpu.VMEM((2, INTER, DIM), jnp.float32),
                pltpu.SemaphoreType.DMA((2, 2)),
            ],
        ),
        out_shape=jax.ShapeDtypeStruct((NP, INTER), jnp.float32),
        compiler_params=pltpu.CompilerParams(
            dimension_semantics=("arbitrary",)),
    )(gmap, valid, posm, nxt, xs, w1, b1[:, None, :], w3, b3[:, None, :])

    outm = pl.pallas_call(
        _stage_b_kernel,
        grid_spec=pltpu.PrefetchScalarGridSpec(
            num_scalar_prefetch=4,
            grid=(W,),
            in_specs=[
                pl.BlockSpec((TM, INTER), lambda m, gm, vl, po, nx: (m, 0)),
                pl.BlockSpec(memory_space=pl.ANY),
                pl.BlockSpec((1, 1, DIM), lambda m, gm, vl, po, nx: (gm[m], 0, 0)),
                pl.BlockSpec((TM, 1), lambda m, gm, vl, po, nx: (m, 0)),
            ],
            out_specs=pl.BlockSpec((TM, DIM), lambda m, gm, vl, po, nx: (m, 0)),
            scratch_shapes=[
                pltpu.VMEM((2, DIM, INTER), jnp.float32),
                pltpu.SemaphoreType.DMA((2, 1)),
            ],
        ),
        out_shape=jax.ShapeDtypeStruct((NP, DIM), jnp.float32),
        compiler_params=pltpu.CompilerParams(
            dimension_semantics=("arbitrary",)),
    )(gmap, valid, posm, nxt, hmid, w2, b2[:, None, :], ws[:, None])

    y = pl.kernel(
        _combine_body,
        out_type=jax.ShapeDtypeStruct((T, DIM), jnp.float32),
        mesh=plsc.VectorSubcoreMesh(core_axis_name="c", subcore_axis_name="s"),
        scratch_types=[
            pltpu.VMEM((CH,), jnp.int32),
            pltpu.VMEM((CH,), jnp.int32),
            pltpu.VMEM((CH, DIM), jnp.float32),
            pltpu.VMEM((CH, DIM), jnp.float32),
            pltpu.SemaphoreType.DMA,
        ],
        compiler_params=pltpu.CompilerParams(needs_layout_passes=False),
    )(outm, dest)
    return y


# speedup vs baseline: 1.0783x; 1.0030x over previous
import jax
import jax.numpy as jnp
from jax import lax
from jax.experimental import pallas as pl
from jax.experimental.pallas import tpu as pltpu
from jax.experimental.pallas import tpu_sc as plsc

T = 2048
DIM = 2048
INTER = 1408
E = 8
TM = 256
NP = T * 2 + E * TM
W = NP // TM
TMG = 256
NW = 32
CH = 16
CPW = T // (NW * CH)


def _gate_kernel(x_ref, gw_ref, meta_ref, tot_ref, carry_ref):
    i = pl.program_id(0)

    @pl.when(i == 0)
    def _init():
        carry_ref[...] = jnp.zeros_like(carry_ref)

    s = jax.nn.sigmoid(
        lax.dot_general(x_ref[...], gw_ref[...], (((1,), (1,)), ((), ())),
                        preferred_element_type=jnp.float32))
    eidx = lax.broadcasted_iota(jnp.int32, s.shape, 1)
    big = jnp.int32(E + 1)
    m1 = jnp.max(s, axis=-1, keepdims=True)
    i1 = jnp.min(jnp.where(s == m1, eidx, big), axis=-1, keepdims=True)
    sel1 = eidx == i1
    s2 = jnp.where(sel1, -jnp.inf, s)
    m2 = jnp.max(s2, axis=-1, keepdims=True)
    i2 = jnp.min(jnp.where(s2 == m2, eidx, big), axis=-1, keepdims=True)
    sel2 = eidx == i2
    denom = m1 + m2
    onehot = jnp.where(sel1 | sel2, 1.0, 0.0)

    ri = lax.broadcasted_iota(jnp.int32, (TMG, TMG), 0)
    ci = lax.broadcasted_iota(jnp.int32, (TMG, TMG), 1)
    ltri = jnp.where(ci < ri, 1.0, 0.0)
    cnt = lax.dot_general(ltri, onehot, (((1,), (0,)), ((), ())),
                          preferred_element_type=jnp.float32) + carry_ref[...]

    ef = eidx.astype(jnp.float32)
    e0 = jnp.sum(jnp.where(sel1, ef, 0.0), axis=-1)
    e1 = jnp.sum(jnp.where(sel2, ef, 0.0), axis=-1)
    c0 = jnp.sum(jnp.where(sel1, cnt, 0.0), axis=-1)
    c1 = jnp.sum(jnp.where(sel2, cnt, 0.0), axis=-1)
    w0 = (m1 / denom)[:, 0]
    w1 = (m2 / denom)[:, 0]
    z = jnp.zeros_like(w0)
    meta_ref[...] = jnp.stack([e0, e1, w0, w1, c0, c1, z, z], axis=0)

    newcarry = carry_ref[...] + jnp.sum(onehot, axis=0, keepdims=True)
    carry_ref[...] = newcarry
    tot_ref[...] = newcarry


def _prefetch_weights(m, gm_ref, pos_ref, nxt_ref, hbm_refs, bufs, sems):
    pos = pos_ref[m]
    slot = lax.rem(pos, 2)

    @pl.when(m == 0)
    def _prime():
        g0 = gm_ref[0]
        nx = nxt_ref[0]
        for k, (hbm, buf) in enumerate(zip(hbm_refs, bufs)):
            pltpu.make_async_copy(hbm.at[g0], buf.at[0], sems.at[0, k]).start()

        @pl.when(nx >= 0)
        def _():
            for k, (hbm, buf) in enumerate(zip(hbm_refs, bufs)):
                pltpu.make_async_copy(hbm.at[nx], buf.at[1],
                                      sems.at[1, k]).start()

    first = (m == 0) | (pos != pos_ref[jnp.maximum(m - 1, 0)])

    @pl.when(first)
    def _wait_issue():
        g = gm_ref[m]
        for k, (hbm, buf) in enumerate(zip(hbm_refs, bufs)):
            pltpu.make_async_copy(hbm.at[g], buf.at[slot],
                                  sems.at[slot, k]).wait()
        nx = nxt_ref[m]

        @pl.when((nx >= 0) & (m != 0))
        def _():
            for k, (hbm, buf) in enumerate(zip(hbm_refs, bufs)):
                pltpu.make_async_copy(hbm.at[nx], buf.at[1 - slot],
                                      sems.at[1 - slot, k]).start()

    return slot


def _stage_a_kernel(gm_ref, vl_ref, pos_ref, nxt_ref, xs_ref, w1_hbm, b1_ref,
                    w3_hbm, b3_ref, h_ref, w1buf, w3buf, sems):
    m = pl.program_id(0)
    slot = _prefetch_weights(m, gm_ref, pos_ref, nxt_ref,
                             [w1_hbm, w3_hbm], [w1buf, w3buf], sems)

    @pl.when(vl_ref[m] == 1)
    def _compute():
        xb = xs_ref[...]
        h1 = lax.dot_general(xb, w1buf[slot], (((1,), (1,)), ((), ())),
                             preferred_element_type=jnp.float32) + b1_ref[0]
        h3 = lax.dot_general(xb, w3buf[slot], (((1,), (1,)), ((), ())),
                             preferred_element_type=jnp.float32) + b3_ref[0]
        h_ref[...] = (h1 * jax.nn.sigmoid(h1)) * h3


def _stage_b_kernel(gm_ref, vl_ref, pos_ref, nxt_ref, h_ref, w2_hbm, b2_ref,
                    ws_ref, out_ref, w2buf, sems):
    m = pl.program_id(0)
    slot = _prefetch_weights(m, gm_ref, pos_ref, nxt_ref,
                             [w2_hbm], [w2buf], sems)

    @pl.when(vl_ref[m] == 1)
    def _compute():
        out_ref[...] = (lax.dot_general(
            h_ref[...], w2buf[slot], (((1,), (1,)), ((), ())),
            preferred_element_type=jnp.float32) + b2_ref[0]) * ws_ref[...]


def _dispatch_body(x_hbm, meta_hbm, aoff_hbm, xs_hbm, dest_hbm, ws_hbm,
                   a_vm, f0, f1, f4, f5, idx0, idx1, rows,
                   eb, cb, wb, wsbuf, sem):
    wid = lax.axis_index("s") * 2 + lax.axis_index("c")
    pltpu.sync_copy(aoff_hbm, a_vm)

    @pl.when(wid == 0)
    def _build_ws():
        for slot in range(2):
            pltpu.sync_copy(meta_hbm.at[slot, :], eb)
            pltpu.sync_copy(meta_hbm.at[4 + slot, :], cb)
            pltpu.sync_copy(meta_hbm.at[2 + slot, :], wb)

            def _scat(j, _):
                ts = pl.ds(j * 16, 16)
                ev = eb[ts].astype(jnp.int32)
                dv = plsc.load_gather(a_vm, [ev]) + cb[ts].astype(jnp.int32)
                plsc.store_scatter(wsbuf, [dv], wb[ts])
                return _

            lax.fori_loop(0, T // 16, _scat, 0)
        pltpu.sync_copy(wsbuf, ws_hbm)

    for ch in range(CPW):
        tok = (wid * CPW + ch) * CH
        pltpu.sync_copy(meta_hbm.at[0, pl.ds(tok, CH)], f0)
        pltpu.sync_copy(meta_hbm.at[1, pl.ds(tok, CH)], f1)
        pltpu.sync_copy(meta_hbm.at[4, pl.ds(tok, CH)], f4)
        pltpu.sync_copy(meta_hbm.at[5, pl.ds(tok, CH)], f5)
        e0 = f0[...].astype(jnp.int32)
        e1 = f1[...].astype(jnp.int32)
        a0 = plsc.load_gather(a_vm, [e0])
        a1 = plsc.load_gather(a_vm, [e1])
        idx0[...] = a0 + f4[...].astype(jnp.int32)
        idx1[...] = a1 + f5[...].astype(jnp.int32)
        pltpu.sync_copy(x_hbm.at[pl.ds(tok, CH)], rows)
        pltpu.async_copy(rows, xs_hbm.at[idx0], sem).wait()
        pltpu.async_copy(rows, xs_hbm.at[idx1], sem).wait()
        pltpu.sync_copy(idx0, dest_hbm.at[0, pl.ds(tok, CH)])
        pltpu.sync_copy(idx1, dest_hbm.at[1, pl.ds(tok, CH)])


def _combine_body(out_hbm, dest_hbm, y_hbm,
                  idx0, idx1, buf0, buf1, sem):
    wid = lax.axis_index("s") * 2 + lax.axis_index("c")
    for ch in range(CPW):
        tok = (wid * CPW + ch) * CH
        pltpu.sync_copy(dest_hbm.at[0, pl.ds(tok, CH)], idx0)
        pltpu.sync_copy(dest_hbm.at[1, pl.ds(tok, CH)], idx1)
        pltpu.async_copy(out_hbm.at[idx0], buf0, sem).wait()
        pltpu.async_copy(out_hbm.at[idx1], buf1, sem).wait()

        def _row(i, _):
            def _col(j, _):
                for u in range(4):
                    cs = pl.ds(j * 64 + u * 16, 16)
                    buf0[i, cs] = buf0[i, cs] + buf1[i, cs]
                return _

            return lax.fori_loop(0, DIM // 64, _col, _)

        lax.fori_loop(0, CH, _row, 0)
        pltpu.sync_copy(buf0, y_hbm.at[pl.ds(tok, CH)])


def kernel(x, gate_w, w1, b1, w2, b2, w3, b3):
    meta, totals = pl.pallas_call(
        _gate_kernel,
        grid=(T // TMG,),
        in_specs=[
            pl.BlockSpec((TMG, DIM), lambda i: (i, 0)),
            pl.BlockSpec((E, DIM), lambda i: (0, 0)),
        ],
        out_specs=[
            pl.BlockSpec((E, TMG), lambda i: (0, i)),
            pl.BlockSpec((1, E), lambda i: (0, 0)),
        ],
        out_shape=[
            jax.ShapeDtypeStruct((E, T), jnp.float32),
            jax.ShapeDtypeStruct((1, E), jnp.float32),
        ],
        scratch_shapes=[pltpu.VMEM((1, E), jnp.float32)],
    )(x, gate_w)

    counts = totals[0].astype(jnp.int32)
    starts = []
    cur = jnp.int32(0)
    for e in range(E):
        starts.append(cur)
        cur = ((cur + counts[e] + TM - 1) // TM) * TM
    aoff = jnp.stack(starts)
    rows0 = jnp.arange(W, dtype=jnp.int32) * TM
    gmap = jnp.clip(jnp.searchsorted(aoff, rows0, side="right").astype(jnp.int32) - 1,
                    0, E - 1)
    valid = (rows0 < aoff[gmap] + counts[gmap]).astype(jnp.int32)
    change = jnp.concatenate([jnp.ones((1,), jnp.int32),
                              (gmap[1:] != gmap[:-1]).astype(jnp.int32)])
    posm = jnp.cumsum(change).astype(jnp.int32) - 1
    nidx = jnp.searchsorted(gmap, gmap, side="right").astype(jnp.int32)
    nxt = jnp.where(nidx < W, gmap[jnp.clip(nidx, 0, W - 1)], -1).astype(jnp.int32)

    mesh = plsc.VectorSubcoreMesh(core_axis_name="c", subcore_axis_name="s")
    xs, dest, ws = pl.kernel(
        _dispatch_body,
        out_type=[
            jax.ShapeDtypeStruct((NP, DIM), jnp.float32),
            jax.ShapeDtypeStruct((2, T), jnp.int32),
            jax.ShapeDtypeStruct((NP,), jnp.float32),
        ],
        mesh=mesh,
        scratch_types=[
            pltpu.VMEM((E,), jnp.int32),
            pltpu.VMEM((CH,), jnp.float32),
            pltpu.VMEM((CH,), jnp.float32),
            pltpu.VMEM((CH,), jnp.float32),
            pltpu.VMEM((CH,), jnp.float32),
            pltpu.VMEM((CH,), jnp.int32),
            pltpu.VMEM((CH,), jnp.int32),
            pltpu.VMEM((CH, DIM), jnp.float32),
            pltpu.VMEM((T,), jnp.float32),
            pltpu.VMEM((T,), jnp.float32),
            pltpu.VMEM((T,), jnp.float32),
            pltpu.VMEM((NP,), jnp.float32),
            pltpu.SemaphoreType.DMA,
        ],
        compiler_params=pltpu.CompilerParams(needs_layout_passes=False),
    )(x, meta, aoff)

    hmid = pl.pallas_call(
        _stage_a_kernel,
        grid_spec=pltpu.PrefetchScalarGridSpec(
            num_scalar_prefetch=4,
            grid=(W,),
            in_specs=[
                pl.BlockSpec((TM, DIM), lambda m, gm, vl, po, nx: (m, 0)),
                pl.BlockSpec(memory_space=pl.ANY),
                pl.BlockSpec((1, 1, INTER), lambda m, gm, vl, po, nx: (gm[m], 0, 0)),
                pl.BlockSpec(memory_space=pl.ANY),
                pl.BlockSpec((1, 1, INTER), lambda m, gm, vl, po, nx: (gm[m], 0, 0)),
            ],
            out_specs=pl.BlockSpec((TM, INTER), lambda m, gm, vl, po, nx: (m, 0)),
            scratch_shapes=[
                pltpu.VMEM((2, INTER, DIM), jnp.float32),
                pltpu.VMEM((2, INTER, DIM), jnp.float32),
                pltpu.SemaphoreType.DMA((2, 2)),
            ],
        ),
        out_shape=jax.ShapeDtypeStruct((NP, INTER), jnp.float32),
        compiler_params=pltpu.CompilerParams(
            dimension_semantics=("arbitrary",)),
    )(gmap, valid, posm, nxt, xs, w1, b1[:, None, :], w3, b3[:, None, :])

    outm = pl.pallas_call(
        _stage_b_kernel,
        grid_spec=pltpu.PrefetchScalarGridSpec(
            num_scalar_prefetch=4,
            grid=(W,),
            in_specs=[
                pl.BlockSpec((TM, INTER), lambda m, gm, vl, po, nx: (m, 0)),
                pl.BlockSpec(memory_space=pl.ANY),
                pl.BlockSpec((1, 1, DIM), lambda m, gm, vl, po, nx: (gm[m], 0, 0)),
                pl.BlockSpec((TM, 1), lambda m, gm, vl, po, nx: (m, 0)),
            ],
            out_specs=pl.BlockSpec((TM, DIM), lambda m, gm, vl, po, nx: (m, 0)),
            scratch_shapes=[
                pltpu.VMEM((2, DIM, INTER), jnp.float32),
                pltpu.SemaphoreType.DMA((2, 1)),
            ],
        ),
        out_shape=jax.ShapeDtypeStruct((NP, DIM), jnp.float32),
        compiler_params=pltpu.CompilerParams(
            dimension_semantics=("arbitrary",)),
    )(gmap, valid, posm, nxt, hmid, w2, b2[:, None, :], ws[:, None])

    y = pl.kernel(
        _combine_body,
        out_type=jax.ShapeDtypeStruct((T, DIM), jnp.float32),
        mesh=plsc.VectorSubcoreMesh(core_axis_name="c", subcore_axis_name="s"),
        scratch_types=[
            pltpu.VMEM((CH,), jnp.int32),
            pltpu.VMEM((CH,), jnp.int32),
            pltpu.VMEM((CH, DIM), jnp.float32),
            pltpu.VMEM((CH, DIM), jnp.float32),
            pltpu.SemaphoreType.DMA,
        ],
        compiler_params=pltpu.CompilerParams(needs_layout_passes=False),
    )(outm, dest)
    return y
